# Initial kernel scaffold; baseline (speedup 1.0000x reference)
import jax
import jax.numpy as jnp
from jax.experimental import pallas as pl


def kernel(query, reference, spatial_size, Wq, bq, Wkv, bkv, Woff, boff, Wo, bo):
    raise NotImplementedError("write your pallas kernel here")


# speedup vs baseline: 2.0873x; 2.0873x over previous
import functools
import math

import jax
import jax.numpy as jnp
from jax import lax
from jax.experimental import pallas as pl
from jax.experimental.pallas import tpu as pltpu
from jax.experimental.pallas import tpu_sc as plsc

DIMC = 384
NH = 8
NP = 8
HD = DIMC // NH
HW = 1024
SIDE = 32
NW = 32
CHUNK = 256
SCALE = HD ** (-0.5)


def _proj_body(x_ref, wq_ref, bq_ref, w0_ref, b0_ref, w1_ref, b1_ref,
               q_ref, px_ref, py_ref, *, tm):
    x = x_ref[...]
    q_ref[...] = (jnp.dot(x, wq_ref[...], preferred_element_type=jnp.float32)
                  + bq_ref[...])
    o0 = jnp.dot(x, w0_ref[...], preferred_element_type=jnp.float32) + b0_ref[...]
    o1 = jnp.dot(x, w1_ref[...], preferred_element_type=jnp.float32) + b1_ref[...]
    row = pl.program_id(0) * tm + lax.broadcasted_iota(jnp.int32, (tm, NH * NP), 0)
    n = row % HW
    gy = (n // SIDE).astype(jnp.float32)
    gx = (n % SIDE).astype(jnp.float32)
    px_ref[...] = gy + o0
    py_ref[...] = gx + o1


def _projections(xflat, Wq, bq, W0, b0, W1, b1):
    rows = xflat.shape[0]
    tm = 512
    grid = rows // tm
    return pl.pallas_call(
        functools.partial(_proj_body, tm=tm),
        grid=(grid,),
        in_specs=[
            pl.BlockSpec((tm, DIMC), lambda i: (i, 0)),
            pl.BlockSpec((DIMC, DIMC), lambda i: (0, 0)),
            pl.BlockSpec((DIMC,), lambda i: (0,)),
            pl.BlockSpec((DIMC, NH * NP), lambda i: (0, 0)),
            pl.BlockSpec((NH * NP,), lambda i: (0,)),
            pl.BlockSpec((DIMC, NH * NP), lambda i: (0, 0)),
            pl.BlockSpec((NH * NP,), lambda i: (0,)),
        ],
        out_specs=[
            pl.BlockSpec((tm, DIMC), lambda i: (i, 0)),
            pl.BlockSpec((tm, NH * NP), lambda i: (i, 0)),
            pl.BlockSpec((tm, NH * NP), lambda i: (i, 0)),
        ],
        out_shape=[
            jax.ShapeDtypeStruct((rows, DIMC), jnp.float32),
            jax.ShapeDtypeStruct((rows, NH * NP), jnp.float32),
            jax.ShapeDtypeStruct((rows, NH * NP), jnp.float32),
        ],
    )(xflat, Wq, bq, W0, b0, W1, b1)


def _sc_body(ref_hbm, px_hbm, py_hbm, q_hbm, out_hbm,
             img_v, px_v, py_v, q_v, o_v, s_v, a_v):
    nc = 2
    w = lax.axis_index("s") * nc + lax.axis_index("c")
    pltpu.sync_copy(ref_hbm.at[w], img_v)

    def chunk_body(ci, _):
        t0 = ci * CHUNK
        pltpu.sync_copy(q_hbm.at[w, :, pl.ds(t0, CHUNK)], q_v)
        pltpu.sync_copy(px_hbm.at[w, :, pl.ds(t0, CHUNK)], px_v)
        pltpu.sync_copy(py_hbm.at[w, :, pl.ds(t0, CHUNK)], py_v)

        def group_body(g, _):
            lb = g * 16
            a_regs = []
            for p in range(NP):
                pxv = px_v[p, pl.ds(lb, 16)]
                pyv = py_v[p, pl.ds(lb, 16)]
                pxv = jnp.clip(pxv, -40.0, 40.0)
                pyv = jnp.clip(pyv, -40.0, 40.0)
                xt = pxv.astype(jnp.int32)
                xtf = xt.astype(jnp.float32)
                xfloor = jnp.where(xtf > pxv, xtf - 1.0, xtf)
                x0i = jnp.where(xtf > pxv, xt - 1, xt)
                yt = pyv.astype(jnp.int32)
                ytf = yt.astype(jnp.float32)
                yfloor = jnp.where(ytf > pyv, ytf - 1.0, ytf)
                y0i = jnp.where(ytf > pyv, yt - 1, yt)
                fx = pxv - xfloor
                fy = pyv - yfloor
                vx0 = (xfloor >= 0.0) & (xfloor <= SIDE - 1.0)
                vx1 = (xfloor + 1.0 >= 0.0) & (xfloor + 1.0 <= SIDE - 1.0)
                vy0 = (yfloor >= 0.0) & (yfloor <= SIDE - 1.0)
                vy1 = (yfloor + 1.0 >= 0.0) & (yfloor + 1.0 <= SIDE - 1.0)
                zero = jnp.zeros((16,), jnp.float32)
                w00 = jnp.where(vy0 & vx0, (1.0 - fx) * (1.0 - fy), zero)
                w10 = jnp.where(vy1 & vx0, (1.0 - fx) * fy, zero)
                w01 = jnp.where(vy0 & vx1, fx * (1.0 - fy), zero)
                w11 = jnp.where(vy1 & vx1, fx * fy, zero)
                xc0 = jnp.clip(x0i, 0, SIDE - 1)
                xc1 = jnp.clip(x0i + 1, 0, SIDE - 1)
                yc0 = jnp.clip(y0i, 0, SIDE - 1)
                yc1 = jnp.clip(y0i + 1, 0, SIDE - 1)
                i00 = yc0 * SIDE + xc0
                i01 = yc0 * SIDE + xc1
                i10 = yc1 * SIDE + xc0
                i11 = yc1 * SIDE + xc1

                def c_body(c, acc):
                    o = c * HW
                    v00 = plsc.load_gather(img_v, [i00 + o])
                    v01 = plsc.load_gather(img_v, [i01 + o])
                    v10 = plsc.load_gather(img_v, [i10 + o])
                    v11 = plsc.load_gather(img_v, [i11 + o])
                    s = v00 * w00 + v01 * w01 + v10 * w10 + v11 * w11
                    s_v[p, pl.ds(c * 16, 16)] = s
                    qc = q_v[c, pl.ds(lb, 16)]
                    return acc + qc * s

                dot = lax.fori_loop(0, HD, c_body,
                                    jnp.zeros((16,), jnp.float32))
                a_regs.append(dot * SCALE)
            m = a_regs[0]
            for p in range(1, NP):
                m = jnp.maximum(m, a_regs[p])
            es = [jnp.exp(a - m) for a in a_regs]
            tot = es[0]
            for p in range(1, NP):
                tot = tot + es[p]
            inv = 1.0 / tot
            aw = [e * inv for e in es]
            for p in range(NP):
                a_v[pl.ds(p * 16, 16)] = aw[p]

            def w_body(c, _):
                acc = a_v[pl.ds(0, 16)] * s_v[0, pl.ds(c * 16, 16)]
                for p in range(1, NP):
                    acc = acc + a_v[pl.ds(p * 16, 16)] * s_v[p, pl.ds(c * 16, 16)]
                o_v[c, pl.ds(lb, 16)] = acc
                return 0

            lax.fori_loop(0, HD, w_body, 0)
            return 0

        lax.fori_loop(0, CHUNK // 16, group_body, 0)
        pltpu.sync_copy(o_v, out_hbm.at[w, :, pl.ds(t0, CHUNK)])
        return 0

    lax.fori_loop(0, HW // CHUNK, chunk_body, 0)


def _sc_sample_attend(ref_sc, pxr, pyr, qT):
    mesh = plsc.VectorSubcoreMesh(core_axis_name="c", subcore_axis_name="s",
                                  num_cores=2, num_subcores=16)
    return pl.kernel(
        _sc_body,
        out_type=jax.ShapeDtypeStruct((NW, HD, HW), jnp.float32),
        mesh=mesh,
        compiler_params=pltpu.CompilerParams(needs_layout_passes=False),
        scratch_types=[
            pltpu.VMEM((HD * HW,), jnp.float32),
            pltpu.VMEM((NP, CHUNK), jnp.float32),
            pltpu.VMEM((NP, CHUNK), jnp.float32),
            pltpu.VMEM((HD, CHUNK), jnp.float32),
            pltpu.VMEM((HD, CHUNK), jnp.float32),
            pltpu.VMEM((NP, HD * 16), jnp.float32),
            pltpu.VMEM((NP * 16,), jnp.float32),
        ],
    )(ref_sc, pxr, pyr, qT)


def _out_body(x_ref, wo_ref, bo_ref, y_ref):
    y_ref[...] = (jnp.dot(x_ref[...], wo_ref[...],
                          preferred_element_type=jnp.float32) + bo_ref[...])


def _out_proj(xflat, Wo, bo):
    rows = xflat.shape[0]
    tm = 512
    return pl.pallas_call(
        _out_body,
        grid=(rows // tm,),
        in_specs=[
            pl.BlockSpec((tm, DIMC), lambda i: (i, 0)),
            pl.BlockSpec((DIMC, DIMC), lambda i: (0, 0)),
            pl.BlockSpec((DIMC,), lambda i: (0,)),
        ],
        out_specs=pl.BlockSpec((tm, DIMC), lambda i: (i, 0)),
        out_shape=jax.ShapeDtypeStruct((rows, DIMC), jnp.float32),
    )(xflat, Wo, bo)


def kernel(query, reference, spatial_size, Wq, bq, Wkv, bkv, Woff, boff, Wo, bo):
    B, N, C = query.shape
    del spatial_size, Wkv, bkv
    W0 = Woff[:, 0::2]
    b0 = boff[0::2]
    W1 = Woff[:, 1::2]
    b1 = boff[1::2]

    q, px, py = _projections(query.reshape(B * N, C), Wq, bq, W0, b0, W1, b1)

    ref_sc = reference.reshape(B * NH, HD * HW)
    pxr = px.reshape(B, N * NH * NP).reshape(B * NH, NP, HW)
    pyr = py.reshape(B, N * NH * NP).reshape(B * NH, NP, HW)
    qT = (q.reshape(B, N, NH, HD).transpose(0, 2, 3, 1)
          .reshape(B * NH, HD, N))

    out_heads = _sc_sample_attend(ref_sc, pxr, pyr, qT)

    out = (out_heads.reshape(B, NH, HD, N).transpose(0, 3, 1, 2)
           .reshape(B * N, C))
    return _out_proj(out, Wo, bo).reshape(B, N, C)


# speedup vs baseline: 3.7846x; 1.8131x over previous
import functools
import math

import jax
import jax.numpy as jnp
from jax import lax
from jax.experimental import pallas as pl
from jax.experimental.pallas import tpu as pltpu
from jax.experimental.pallas import tpu_sc as plsc

DIMC = 384
NH = 8
NP = 8
HD = DIMC // NH
HD2 = HD // 2
HW = 1024
SIDE = 32
NW = 32
CHUNK = 256
SCALE = HD ** (-0.5)


def _proj_body(x_ref, wq_ref, bq_ref, w0_ref, b0_ref, w1_ref, b1_ref,
               q_ref, px_ref, py_ref, *, tm):
    x = x_ref[...]
    q_ref[...] = (jnp.dot(x, wq_ref[...], preferred_element_type=jnp.float32)
                  + bq_ref[...])
    o0 = jnp.dot(x, w0_ref[...], preferred_element_type=jnp.float32) + b0_ref[...]
    o1 = jnp.dot(x, w1_ref[...], preferred_element_type=jnp.float32) + b1_ref[...]
    row = pl.program_id(0) * tm + lax.broadcasted_iota(jnp.int32, (tm, NH * NP), 0)
    n = row % HW
    gy = (n // SIDE).astype(jnp.float32)
    gx = (n % SIDE).astype(jnp.float32)
    px_ref[...] = gy + o0
    py_ref[...] = gx + o1


def _projections(xflat, Wq, bq, W0, b0, W1, b1):
    rows = xflat.shape[0]
    tm = 512
    grid = rows // tm
    return pl.pallas_call(
        functools.partial(_proj_body, tm=tm),
        grid=(grid,),
        in_specs=[
            pl.BlockSpec((tm, DIMC), lambda i: (i, 0)),
            pl.BlockSpec((DIMC, DIMC), lambda i: (0, 0)),
            pl.BlockSpec((DIMC,), lambda i: (0,)),
            pl.BlockSpec((DIMC, NH * NP), lambda i: (0, 0)),
            pl.BlockSpec((NH * NP,), lambda i: (0,)),
            pl.BlockSpec((DIMC, NH * NP), lambda i: (0, 0)),
            pl.BlockSpec((NH * NP,), lambda i: (0,)),
        ],
        out_specs=[
            pl.BlockSpec((tm, DIMC), lambda i: (i, 0)),
            pl.BlockSpec((tm, NH * NP), lambda i: (i, 0)),
            pl.BlockSpec((tm, NH * NP), lambda i: (i, 0)),
        ],
        out_shape=[
            jax.ShapeDtypeStruct((rows, DIMC), jnp.float32),
            jax.ShapeDtypeStruct((rows, NH * NP), jnp.float32),
            jax.ShapeDtypeStruct((rows, NH * NP), jnp.float32),
        ],
    )(xflat, Wq, bq, W0, b0, W1, b1)


def _pack_body(lo_ref, hi_ref, out_ref):
    lo = lax.bitcast_convert_type(lo_ref[...].astype(jnp.bfloat16), jnp.uint16)
    hi = lax.bitcast_convert_type(hi_ref[...].astype(jnp.bfloat16), jnp.uint16)
    word = lo.astype(jnp.int32) | (hi.astype(jnp.int32) << 16)
    out_ref[...] = word


def _pack_pairs(lo, hi):
    return pl.pallas_call(
        _pack_body,
        grid=(4,),
        in_specs=[
            pl.BlockSpec((8, HD2, HW), lambda i: (i, 0, 0)),
            pl.BlockSpec((8, HD2, HW), lambda i: (i, 0, 0)),
        ],
        out_specs=pl.BlockSpec((8, HD2, HW), lambda i: (i, 0, 0)),
        out_shape=jax.ShapeDtypeStruct((NW, HD2, HW), jnp.int32),
    )(lo, hi)


def _sc_body(ref_hbm, px_hbm, py_hbm, q_hbm, out_hbm,
             img_v, px_v, py_v, q_v, o_v, s_v):
    nc = 2
    w = lax.axis_index("s") * nc + lax.axis_index("c")
    pltpu.sync_copy(ref_hbm.at[w], img_v)

    def chunk_body(ci, _):
        t0 = ci * CHUNK
        pltpu.sync_copy(q_hbm.at[w, :, pl.ds(t0, CHUNK)], q_v)
        pltpu.sync_copy(px_hbm.at[w, :, pl.ds(t0, CHUNK)], px_v)
        pltpu.sync_copy(py_hbm.at[w, :, pl.ds(t0, CHUNK)], py_v)

        def group_body(g, _):
            lb = g * 16
            logits = [None] * NP
            for half in range(2):
                bases = []
                weights = []
                for p in range(4 * half, 4 * half + 4):
                    pxv = px_v[p, pl.ds(lb, 16)]
                    pyv = py_v[p, pl.ds(lb, 16)]
                    pxv = jnp.clip(pxv, -40.0, 40.0)
                    pyv = jnp.clip(pyv, -40.0, 40.0)
                    xt = pxv.astype(jnp.int32)
                    xtf = xt.astype(jnp.float32)
                    xfloor = jnp.where(xtf > pxv, xtf - 1.0, xtf)
                    x0i = jnp.where(xtf > pxv, xt - 1, xt)
                    yt = pyv.astype(jnp.int32)
                    ytf = yt.astype(jnp.float32)
                    yfloor = jnp.where(ytf > pyv, ytf - 1.0, ytf)
                    y0i = jnp.where(ytf > pyv, yt - 1, yt)
                    fx = pxv - xfloor
                    fy = pyv - yfloor
                    vx0 = (xfloor >= 0.0) & (xfloor <= SIDE - 1.0)
                    vx1 = (xfloor + 1.0 >= 0.0) & (xfloor + 1.0 <= SIDE - 1.0)
                    vy0 = (yfloor >= 0.0) & (yfloor <= SIDE - 1.0)
                    vy1 = (yfloor + 1.0 >= 0.0) & (yfloor + 1.0 <= SIDE - 1.0)
                    zero = jnp.zeros((16,), jnp.float32)
                    w00 = jnp.where(vy0 & vx0, (1.0 - fx) * (1.0 - fy), zero)
                    w10 = jnp.where(vy1 & vx0, (1.0 - fx) * fy, zero)
                    w01 = jnp.where(vy0 & vx1, fx * (1.0 - fy), zero)
                    w11 = jnp.where(vy1 & vx1, fx * fy, zero)
                    xc0 = jnp.clip(x0i, 0, SIDE - 1)
                    xc1 = jnp.clip(x0i + 1, 0, SIDE - 1)
                    yc0 = jnp.clip(y0i, 0, SIDE - 1)
                    yc1 = jnp.clip(y0i + 1, 0, SIDE - 1)
                    bases.append((yc0 * SIDE + xc0, yc0 * SIDE + xc1,
                                  yc1 * SIDE + xc0, yc1 * SIDE + xc1))
                    weights.append((w00, w01, w10, w11))

                def c_body(c2, accs):
                    o = c2 * HW
                    qlo = q_v[2 * c2, pl.ds(lb, 16)]
                    qhi = q_v[2 * c2 + 1, pl.ds(lb, 16)]
                    out_accs = []
                    for k in range(4):
                        i00, i01, i10, i11 = bases[k]
                        w00, w01, w10, w11 = weights[k]
                        g00 = plsc.load_gather(img_v, [i00 + o])
                        g01 = plsc.load_gather(img_v, [i01 + o])
                        g10 = plsc.load_gather(img_v, [i10 + o])
                        g11 = plsc.load_gather(img_v, [i11 + o])
                        s_lo = (w00 * plsc.bitcast(g00 << 16, jnp.float32)
                                + w01 * plsc.bitcast(g01 << 16, jnp.float32)
                                + w10 * plsc.bitcast(g10 << 16, jnp.float32)
                                + w11 * plsc.bitcast(g11 << 16, jnp.float32))
                        s_hi = (w00 * plsc.bitcast(g00, jnp.float32)
                                + w01 * plsc.bitcast(g01, jnp.float32)
                                + w10 * plsc.bitcast(g10, jnp.float32)
                                + w11 * plsc.bitcast(g11, jnp.float32))
                        p = 4 * half + k
                        s_v[p, pl.ds(32 * c2, 16)] = s_lo
                        s_v[p, pl.ds(32 * c2 + 16, 16)] = s_hi
                        out_accs.append(accs[k] + qlo * s_lo + qhi * s_hi)
                    return tuple(out_accs)

                zeros4 = (jnp.zeros((16,), jnp.float32),) * 4
                accs = plsc.parallel_loop(0, HD2, unroll=4, carry=zeros4)(c_body)
                for k in range(4):
                    logits[4 * half + k] = accs[k] * SCALE

            m = logits[0]
            for p in range(1, NP):
                m = jnp.maximum(m, logits[p])
            es = [jnp.exp(a - m) for a in logits]
            tot = es[0]
            for p in range(1, NP):
                tot = tot + es[p]
            inv = 1.0 / tot
            aw = [e * inv for e in es]

            def w_body(c):
                acc = aw[0] * s_v[0, pl.ds(c * 16, 16)]
                for p in range(1, NP):
                    acc = acc + aw[p] * s_v[p, pl.ds(c * 16, 16)]
                o_v[c, pl.ds(lb, 16)] = acc

            plsc.parallel_loop(0, HD, unroll=4)(w_body)
            return 0

        lax.fori_loop(0, CHUNK // 16, group_body, 0)
        pltpu.sync_copy(o_v, out_hbm.at[w, :, pl.ds(t0, CHUNK)])
        return 0

    lax.fori_loop(0, HW // CHUNK, chunk_body, 0)


def _sc_sample_attend(ref_packed, pxr, pyr, qT):
    mesh = plsc.VectorSubcoreMesh(core_axis_name="c", subcore_axis_name="s",
                                  num_cores=2, num_subcores=16)
    return pl.kernel(
        _sc_body,
        out_type=jax.ShapeDtypeStruct((NW, HD, HW), jnp.float32),
        mesh=mesh,
        compiler_params=pltpu.CompilerParams(needs_layout_passes=False),
        scratch_types=[
            pltpu.VMEM((HD2 * HW,), jnp.int32),
            pltpu.VMEM((NP, CHUNK), jnp.float32),
            pltpu.VMEM((NP, CHUNK), jnp.float32),
            pltpu.VMEM((HD, CHUNK), jnp.float32),
            pltpu.VMEM((HD, CHUNK), jnp.float32),
            pltpu.VMEM((NP, HD * 16), jnp.float32),
        ],
    )(ref_packed, pxr, pyr, qT)


def _out_body(x_ref, wo_ref, bo_ref, y_ref):
    y_ref[...] = (jnp.dot(x_ref[...], wo_ref[...],
                          preferred_element_type=jnp.float32) + bo_ref[...])


def _out_proj(xflat, Wo, bo):
    rows = xflat.shape[0]
    tm = 512
    return pl.pallas_call(
        _out_body,
        grid=(rows // tm,),
        in_specs=[
            pl.BlockSpec((tm, DIMC), lambda i: (i, 0)),
            pl.BlockSpec((DIMC, DIMC), lambda i: (0, 0)),
            pl.BlockSpec((DIMC,), lambda i: (0,)),
        ],
        out_specs=pl.BlockSpec((tm, DIMC), lambda i: (i, 0)),
        out_shape=jax.ShapeDtypeStruct((rows, DIMC), jnp.float32),
    )(xflat, Wo, bo)


def kernel(query, reference, spatial_size, Wq, bq, Wkv, bkv, Woff, boff, Wo, bo):
    B, N, C = query.shape
    del spatial_size, Wkv, bkv
    W0 = Woff[:, 0::2]
    b0 = boff[0::2]
    W1 = Woff[:, 1::2]
    b1 = boff[1::2]

    q, px, py = _projections(query.reshape(B * N, C), Wq, bq, W0, b0, W1, b1)

    ref_pairs = reference.reshape(B * NH, HD2, 2, HW)
    ref_packed = _pack_pairs(ref_pairs[:, :, 0],
                             ref_pairs[:, :, 1]).reshape(NW, HD2 * HW)
    pxr = px.reshape(B, N * NH * NP).reshape(B * NH, NP, HW)
    pyr = py.reshape(B, N * NH * NP).reshape(B * NH, NP, HW)
    qT = (q.reshape(B, N, NH, HD).transpose(0, 2, 3, 1)
          .reshape(B * NH, HD, N))

    out_heads = _sc_sample_attend(ref_packed, pxr, pyr, qT)

    out = (out_heads.reshape(B, NH, HD, N).transpose(0, 3, 1, 2)
           .reshape(B * N, C))
    return _out_proj(out, Wo, bo).reshape(B, N, C)


# speedup vs baseline: 3.9794x; 1.0515x over previous
import functools
import math

import jax
import jax.numpy as jnp
from jax import lax
from jax.experimental import pallas as pl
from jax.experimental.pallas import tpu as pltpu
from jax.experimental.pallas import tpu_sc as plsc

DIMC = 384
NH = 8
NP = 8
HD = DIMC // NH
HD2 = HD // 2
HW = 1024
SIDE = 32
NW = 32
CHUNK = 256
SCALE = HD ** (-0.5)


def _proj_body(x_ref, wq_ref, bq_ref, w0_ref, b0_ref, w1_ref, b1_ref,
               lo_ref, hi_ref, qt_ref, px_ref, py_ref, pk_ref, *, tm):
    x = x_ref[...]
    q = jnp.dot(x, wq_ref[...], preferred_element_type=jnp.float32) + bq_ref[...]
    qt_ref[...] = q.T
    o0 = jnp.dot(x, w0_ref[...], preferred_element_type=jnp.float32) + b0_ref[...]
    o1 = jnp.dot(x, w1_ref[...], preferred_element_type=jnp.float32) + b1_ref[...]
    row = pl.program_id(0) * tm + lax.broadcasted_iota(jnp.int32, (tm, NH * NP), 0)
    n = row % HW
    gy = (n // SIDE).astype(jnp.float32)
    gx = (n % SIDE).astype(jnp.float32)
    px_ref[...] = gy + o0
    py_ref[...] = gx + o1
    lo = lax.bitcast_convert_type(lo_ref[...].astype(jnp.bfloat16), jnp.uint16)
    hi = lax.bitcast_convert_type(hi_ref[...].astype(jnp.bfloat16), jnp.uint16)
    pk_ref[...] = lo.astype(jnp.int32) | (hi.astype(jnp.int32) << 16)


def _projections(xflat, Wq, bq, W0, b0, W1, b1, lo, hi):
    rows = xflat.shape[0]
    tm = 512
    grid = rows // tm
    return pl.pallas_call(
        functools.partial(_proj_body, tm=tm),
        grid=(grid,),
        in_specs=[
            pl.BlockSpec((tm, DIMC), lambda i: (i, 0)),
            pl.BlockSpec((DIMC, DIMC), lambda i: (0, 0)),
            pl.BlockSpec((DIMC,), lambda i: (0,)),
            pl.BlockSpec((DIMC, NH * NP), lambda i: (0, 0)),
            pl.BlockSpec((NH * NP,), lambda i: (0,)),
            pl.BlockSpec((DIMC, NH * NP), lambda i: (0, 0)),
            pl.BlockSpec((NH * NP,), lambda i: (0,)),
            pl.BlockSpec((4, HD2, HW), lambda i: (i, 0, 0)),
            pl.BlockSpec((4, HD2, HW), lambda i: (i, 0, 0)),
        ],
        out_specs=[
            pl.BlockSpec((DIMC, tm), lambda i: (i // 2, i % 2)),
            pl.BlockSpec((tm, NH * NP), lambda i: (i, 0)),
            pl.BlockSpec((tm, NH * NP), lambda i: (i, 0)),
            pl.BlockSpec((4, HD2, HW), lambda i: (i, 0, 0)),
        ],
        out_shape=[
            jax.ShapeDtypeStruct((4 * DIMC, rows // 4), jnp.float32),
            jax.ShapeDtypeStruct((rows, NH * NP), jnp.float32),
            jax.ShapeDtypeStruct((rows, NH * NP), jnp.float32),
            jax.ShapeDtypeStruct((NW, HD2, HW), jnp.int32),
        ],
    )(xflat, Wq, bq, W0, b0, W1, b1, lo, hi)


def _sc_body(ref_hbm, px_hbm, py_hbm, q_hbm, out_hbm,
             img_v, px_v, py_v, q_v, o_v, s_v):
    nc = 2
    w = lax.axis_index("s") * nc + lax.axis_index("c")
    pltpu.sync_copy(ref_hbm.at[w], img_v)

    def chunk_body(ci, _):
        t0 = ci * CHUNK
        pltpu.sync_copy(q_hbm.at[w, :, pl.ds(t0, CHUNK)], q_v)
        pltpu.sync_copy(px_hbm.at[w, :, pl.ds(t0, CHUNK)], px_v)
        pltpu.sync_copy(py_hbm.at[w, :, pl.ds(t0, CHUNK)], py_v)

        def group_body(g, _):
            lb = g * 16
            logits = [None] * NP
            for half in range(2):
                bases = []
                weights = []
                for p in range(4 * half, 4 * half + 4):
                    pxv = px_v[p, pl.ds(lb, 16)]
                    pyv = py_v[p, pl.ds(lb, 16)]
                    pxv = jnp.clip(pxv, -40.0, 40.0)
                    pyv = jnp.clip(pyv, -40.0, 40.0)
                    xt = pxv.astype(jnp.int32)
                    xtf = xt.astype(jnp.float32)
                    xfloor = jnp.where(xtf > pxv, xtf - 1.0, xtf)
                    x0i = jnp.where(xtf > pxv, xt - 1, xt)
                    yt = pyv.astype(jnp.int32)
                    ytf = yt.astype(jnp.float32)
                    yfloor = jnp.where(ytf > pyv, ytf - 1.0, ytf)
                    y0i = jnp.where(ytf > pyv, yt - 1, yt)
                    fx = pxv - xfloor
                    fy = pyv - yfloor
                    vx0 = (xfloor >= 0.0) & (xfloor <= SIDE - 1.0)
                    vx1 = (xfloor + 1.0 >= 0.0) & (xfloor + 1.0 <= SIDE - 1.0)
                    vy0 = (yfloor >= 0.0) & (yfloor <= SIDE - 1.0)
                    vy1 = (yfloor + 1.0 >= 0.0) & (yfloor + 1.0 <= SIDE - 1.0)
                    zero = jnp.zeros((16,), jnp.float32)
                    w00 = jnp.where(vy0 & vx0, (1.0 - fx) * (1.0 - fy), zero)
                    w10 = jnp.where(vy1 & vx0, (1.0 - fx) * fy, zero)
                    w01 = jnp.where(vy0 & vx1, fx * (1.0 - fy), zero)
                    w11 = jnp.where(vy1 & vx1, fx * fy, zero)
                    xc0 = jnp.clip(x0i, 0, SIDE - 1)
                    xc1 = jnp.clip(x0i + 1, 0, SIDE - 1)
                    yc0 = jnp.clip(y0i, 0, SIDE - 1)
                    yc1 = jnp.clip(y0i + 1, 0, SIDE - 1)
                    bases.append((xc0 * SIDE + yc0, xc1 * SIDE + yc0,
                                  xc0 * SIDE + yc1, xc1 * SIDE + yc1))
                    weights.append((w00, w01, w10, w11))

                def c_body(c2, accs):
                    o = c2 * HW
                    qlo = q_v[2 * c2, pl.ds(lb, 16)]
                    qhi = q_v[2 * c2 + 1, pl.ds(lb, 16)]
                    out_accs = []
                    for k in range(4):
                        i00, i01, i10, i11 = bases[k]
                        w00, w01, w10, w11 = weights[k]
                        g00 = plsc.load_gather(img_v, [i00 + o])
                        g01 = plsc.load_gather(img_v, [i01 + o])
                        g10 = plsc.load_gather(img_v, [i10 + o])
                        g11 = plsc.load_gather(img_v, [i11 + o])
                        s_lo = (w00 * plsc.bitcast(g00 << 16, jnp.float32)
                                + w01 * plsc.bitcast(g01 << 16, jnp.float32)
                                + w10 * plsc.bitcast(g10 << 16, jnp.float32)
                                + w11 * plsc.bitcast(g11 << 16, jnp.float32))
                        s_hi = (w00 * plsc.bitcast(g00, jnp.float32)
                                + w01 * plsc.bitcast(g01, jnp.float32)
                                + w10 * plsc.bitcast(g10, jnp.float32)
                                + w11 * plsc.bitcast(g11, jnp.float32))
                        p = 4 * half + k
                        s_v[p, pl.ds(32 * c2, 16)] = s_lo
                        s_v[p, pl.ds(32 * c2 + 16, 16)] = s_hi
                        out_accs.append(accs[k] + qlo * s_lo + qhi * s_hi)
                    return tuple(out_accs)

                zeros4 = (jnp.zeros((16,), jnp.float32),) * 4
                accs = plsc.parallel_loop(0, HD2, unroll=8, carry=zeros4)(c_body)
                for k in range(4):
                    logits[4 * half + k] = accs[k] * SCALE

            m = logits[0]
            for p in range(1, NP):
                m = jnp.maximum(m, logits[p])
            es = [jnp.exp(a - m) for a in logits]
            tot = es[0]
            for p in range(1, NP):
                tot = tot + es[p]
            inv = 1.0 / tot
            aw = [e * inv for e in es]

            def w_body(c):
                acc = aw[0] * s_v[0, pl.ds(c * 16, 16)]
                for p in range(1, NP):
                    acc = acc + aw[p] * s_v[p, pl.ds(c * 16, 16)]
                o_v[c, pl.ds(lb, 16)] = acc

            plsc.parallel_loop(0, HD, unroll=8)(w_body)
            return 0

        lax.fori_loop(0, CHUNK // 16, group_body, 0)
        pltpu.sync_copy(o_v, out_hbm.at[w, :, pl.ds(t0, CHUNK)])
        return 0

    lax.fori_loop(0, HW // CHUNK, chunk_body, 0)


def _sc_sample_attend(ref_packed, pxr, pyr, qT):
    mesh = plsc.VectorSubcoreMesh(core_axis_name="c", subcore_axis_name="s",
                                  num_cores=2, num_subcores=16)
    return pl.kernel(
        _sc_body,
        out_type=jax.ShapeDtypeStruct((NW, HD, HW), jnp.float32),
        mesh=mesh,
        compiler_params=pltpu.CompilerParams(needs_layout_passes=False),
        scratch_types=[
            pltpu.VMEM((HD2 * HW,), jnp.int32),
            pltpu.VMEM((NP, CHUNK), jnp.float32),
            pltpu.VMEM((NP, CHUNK), jnp.float32),
            pltpu.VMEM((HD, CHUNK), jnp.float32),
            pltpu.VMEM((HD, CHUNK), jnp.float32),
            pltpu.VMEM((NP, HD * 16), jnp.float32),
        ],
    )(ref_packed, pxr, pyr, qT)


def _out_body(xt_ref, wo_ref, bo_ref, y_ref):
    y_ref[...] = (lax.dot_general(xt_ref[...], wo_ref[...],
                                  (((0,), (0,)), ((), ())),
                                  preferred_element_type=jnp.float32)
                  + bo_ref[...])


def _out_proj(xT, Wo, bo, rows):
    tm = 512
    return pl.pallas_call(
        _out_body,
        grid=(rows // tm,),
        in_specs=[
            pl.BlockSpec((DIMC, tm), lambda i: (i // 2, i % 2)),
            pl.BlockSpec((DIMC, DIMC), lambda i: (0, 0)),
            pl.BlockSpec((DIMC,), lambda i: (0,)),
        ],
        out_specs=pl.BlockSpec((tm, DIMC), lambda i: (i, 0)),
        out_shape=jax.ShapeDtypeStruct((rows, DIMC), jnp.float32),
    )(xT, Wo, bo)


def kernel(query, reference, spatial_size, Wq, bq, Wkv, bkv, Woff, boff, Wo, bo):
    B, N, C = query.shape
    del spatial_size, Wkv, bkv
    W0 = Woff[:, 0::2]
    b0 = boff[0::2]
    W1 = Woff[:, 1::2]
    b1 = boff[1::2]

    ref_pairs = reference.reshape(B * NH, HD2, 2, SIDE, SIDE)
    lo_t = ref_pairs[:, :, 0].swapaxes(-1, -2).reshape(B * NH, HD2, HW)
    hi_t = ref_pairs[:, :, 1].swapaxes(-1, -2).reshape(B * NH, HD2, HW)
    qt2, px, py, ref_packed = _projections(
        query.reshape(B * N, C), Wq, bq, W0, b0, W1, b1, lo_t, hi_t)

    pxr = px.reshape(B, N * NH * NP).reshape(B * NH, NP, HW)
    pyr = py.reshape(B, N * NH * NP).reshape(B * NH, NP, HW)
    qT = qt2.reshape(B * NH, HD, N)
    ref_flat = ref_packed.reshape(NW, HD2 * HW)

    out_heads = _sc_sample_attend(ref_flat, pxr, pyr, qT)

    xT = out_heads.reshape(B * C, N)
    return _out_proj(xT, Wo, bo, B * N).reshape(B, N, C)
